# Initial kernel scaffold; baseline (speedup 1.0000x reference)
import functools

import jax
import jax.numpy as jnp
from jax import lax
from jax.experimental import pallas as pl
from jax.experimental.pallas import tpu as pltpu
from jax.experimental.pallas import tpu_sc as plsc

N = 10000
E = 320000
D = 128
O = 64
ACC_ROWS = 10240
EPAD_ROWS = 2560
CW = 128
NB = 4

_MESH = dict(core_axis_name="c", subcore_axis_name="s", num_cores=2,
             num_subcores=16)


def _pad_edges(src, dst):
    pad = EPAD_ROWS * CW - E
    ar = jnp.arange(pad, dtype=jnp.int32)
    src_p = jnp.concatenate([src, ar % N]).reshape(EPAD_ROWS, CW)
    dst_p = jnp.concatenate([dst, N + (ar % (ACC_ROWS - N))]).reshape(
        EPAD_ROWS, CW)
    return src_p, dst_p


def _zero_buf(rows):
    z = jnp.zeros((16,), jnp.float32)

    def body(r, carry):
        for q in range(8):
            rows[0, r, pl.ds(q * 16, 16)] = z
        return carry

    lax.fori_loop(0, 128, body, 0)


def _zero_acc_stripe(rows, acc, s):
    for t in range(5):
        pltpu.sync_copy(rows.at[0], acc.at[pl.ds(s * 640 + t * 128, 128)])


def _scale_rows(rows, b, wbuf, j):
    def grp(g, carry):
        for i in range(16):
            r = g * 16 + i
            idx0 = jnp.broadcast_to(j, (16,)).astype(jnp.int32)
            idx1 = jnp.broadcast_to(r, (16,)).astype(jnp.int32)
            wb = plsc.load_gather(wbuf, [idx0, idx1])
            for q in range(8):
                sl = pl.ds(q * 16, 16)
                rows[b, r, sl] = rows[b, r, sl] * wb
        return carry

    lax.fori_loop(0, 8, grp, 0)


def _edge_loop(x_hbm, src_idx, dst_idx, rows, acc, sems, n_chunks, scale_fn):
    for b in range(NB):
        pltpu.async_copy(x_hbm.at[src_idx.at[b]], rows.at[b], sems.at[b])

    def outer(jo, carry):
        for b in range(NB):
            j = jo * NB + b
            pltpu.make_async_copy(
                x_hbm.at[src_idx.at[j]], rows.at[b], sems.at[b]).wait()
            scale_fn(rows, b, j)
            pltpu.sync_copy(rows.at[b], acc.at[dst_idx.at[j]], add=True)

            @pl.when(j + NB < n_chunks)
            def _():
                pltpu.async_copy(
                    x_hbm.at[src_idx.at[j + NB]], rows.at[b], sems.at[b])
        return carry

    lax.fori_loop(0, n_chunks // NB, outer, 0)


@functools.partial(
    pl.kernel,
    out_type=jax.ShapeDtypeStruct((2, N, D), jnp.float32),
    mesh=plsc.VectorSubcoreMesh(**_MESH),
    scratch_types=[
        pltpu.VMEM((EPAD_ROWS // 16, CW), jnp.int32),
        pltpu.VMEM((EPAD_ROWS // 16, CW), jnp.int32),
        pltpu.VMEM((EPAD_ROWS // 16, CW), jnp.float32),
        pltpu.VMEM((NB, CW, D), jnp.float32),
        pltpu.VMEM_SHARED((ACC_ROWS, D), jnp.float32),
        pltpu.SemaphoreType.DMA((NB,)),
    ],
)
def _sc_conv12(x_hbm, src_hbm, dst_hbm, w_hbm, out_hbm,
               src_idx, dst_idx, wbuf, rows, acc, sems):
    c = lax.axis_index("c")
    s = lax.axis_index("s")
    n_chunks = EPAD_ROWS // 16

    _zero_buf(rows)
    _zero_acc_stripe(rows, acc, s)

    base = s * n_chunks
    pltpu.sync_copy(src_hbm.at[c, pl.ds(base, n_chunks)], src_idx)
    pltpu.sync_copy(dst_hbm.at[c, pl.ds(base, n_chunks)], dst_idx)

    @pl.when(c == 0)
    def _():
        pltpu.sync_copy(w_hbm.at[pl.ds(base, n_chunks)], wbuf)

    plsc.subcore_barrier()

    def scale_fn(rows_, b, j):
        @pl.when(c == 0)
        def _():
            _scale_rows(rows_, b, wbuf, j)

    _edge_loop(x_hbm, src_idx, dst_idx, rows, acc, sems, n_chunks, scale_fn)

    plsc.subcore_barrier()
    pltpu.sync_copy(acc.at[pl.ds(s * 625, 625)],
                    out_hbm.at[c, pl.ds(s * 625, 625)])


@functools.partial(
    pl.kernel,
    out_type=jax.ShapeDtypeStruct((2, N, D), jnp.float32),
    mesh=plsc.VectorSubcoreMesh(**_MESH),
    scratch_types=[
        pltpu.VMEM((EPAD_ROWS // 32, CW), jnp.int32),
        pltpu.VMEM((EPAD_ROWS // 32, CW), jnp.int32),
        pltpu.VMEM((NB, CW, D), jnp.float32),
        pltpu.VMEM_SHARED((ACC_ROWS, D), jnp.float32),
        pltpu.SemaphoreType.DMA((NB,)),
    ],
)
def _sc_conv3(x_hbm, src_hbm, dst_hbm, out_hbm,
              src_idx, dst_idx, rows, acc, sems):
    c = lax.axis_index("c")
    s = lax.axis_index("s")
    n_chunks = EPAD_ROWS // 32

    _zero_buf(rows)
    _zero_acc_stripe(rows, acc, s)

    base = (c * 16 + s) * n_chunks
    pltpu.sync_copy(src_hbm.at[pl.ds(base, n_chunks)], src_idx)
    pltpu.sync_copy(dst_hbm.at[pl.ds(base, n_chunks)], dst_idx)

    plsc.subcore_barrier()

    _edge_loop(x_hbm, src_idx, dst_idx, rows, acc, sems, n_chunks,
               lambda rows_, b, j: None)

    plsc.subcore_barrier()
    pltpu.sync_copy(acc.at[pl.ds(s * 625, 625)],
                    out_hbm.at[c, pl.ds(s * 625, 625)])


def _tc_combine2(agg12, x_meas, x_dem, W_rel1, b_rel1, W_root1,
                 W_rel2, b_rel2, W_root2):
    BR = 500
    grid = (N // BR,)

    def body(agg_ref, xm_ref, xd_ref, wr1_ref, b1_ref, wo1_ref,
             wr2_ref, b2_ref, wo2_ref, mov_ref, usr_ref):
        f32 = jnp.float32
        a1 = agg_ref[0]
        a2 = agg_ref[1]
        m = (jnp.dot(a1, wr1_ref[...], preferred_element_type=f32)
             + b1_ref[...]
             + jnp.dot(xm_ref[...], wo1_ref[...], preferred_element_type=f32))
        u = (jnp.dot(a2, wr2_ref[...], preferred_element_type=f32)
             + b2_ref[...]
             + jnp.dot(xd_ref[...], wo2_ref[...], preferred_element_type=f32))
        mov_ref[...] = jnp.maximum(m, 0.0)
        usr_ref[...] = jnp.maximum(u, 0.0)

    full = lambda shape: pl.BlockSpec(shape, lambda i: (0,) * len(shape))
    return pl.pallas_call(
        body,
        grid=grid,
        in_specs=[
            pl.BlockSpec((2, BR, D), lambda i: (0, i, 0)),
            pl.BlockSpec((BR, D), lambda i: (i, 0)),
            pl.BlockSpec((BR, D), lambda i: (i, 0)),
            full((D, D)), full((1, D)), full((D, D)),
            full((D, D)), full((1, D)), full((D, D)),
        ],
        out_specs=[pl.BlockSpec((BR, D), lambda i: (i, 0)),
                   pl.BlockSpec((BR, D), lambda i: (i, 0))],
        out_shape=[jax.ShapeDtypeStruct((N, D), jnp.float32),
                   jax.ShapeDtypeStruct((N, D), jnp.float32)],
    )(agg12, x_meas, x_dem, W_rel1, b_rel1.reshape(1, D), W_root1,
      W_rel2, b_rel2.reshape(1, D), W_root2)


def _tc_combine3(p3, user_x1, W_rel3, b_rel3, W_root3, W_lin, b_lin):
    BR = 500
    grid = (N // BR,)

    def body(p3_ref, u1_ref, wr3_ref, b3_ref, wo3_ref, wl_ref, bl_ref,
             out_ref):
        f32 = jnp.float32
        agg3 = p3_ref[0] + p3_ref[1]
        u = (jnp.dot(agg3, wr3_ref[...], preferred_element_type=f32)
             + b3_ref[...]
             + jnp.dot(u1_ref[...], wo3_ref[...], preferred_element_type=f32))
        u = jnp.maximum(u, 0.0)
        out_ref[...] = (jnp.dot(u, wl_ref[...], preferred_element_type=f32)
                        + bl_ref[...])

    full = lambda shape: pl.BlockSpec(shape, lambda i: (0,) * len(shape))
    return pl.pallas_call(
        body,
        grid=grid,
        in_specs=[
            pl.BlockSpec((2, BR, D), lambda i: (0, i, 0)),
            pl.BlockSpec((BR, D), lambda i: (i, 0)),
            full((D, D)), full((1, D)), full((D, D)),
            full((D, O)), full((1, O)),
        ],
        out_specs=pl.BlockSpec((BR, O), lambda i: (i, 0)),
        out_shape=jax.ShapeDtypeStruct((N, O), jnp.float32),
    )(p3, user_x1, W_rel3, b_rel3.reshape(1, D), W_root3,
      W_lin, b_lin.reshape(1, O))


def kernel(x_measurement, x_demand, edge_index_mp, edge_index_rev,
           edge_weight, W_rel1, b_rel1, W_root1, W_rel2, b_rel2, W_root2,
           W_rel3, b_rel3, W_root3, W_lin, b_lin):
    src_mp, dst_mp = _pad_edges(edge_index_mp[0], edge_index_mp[1])
    src_rv, dst_rv = _pad_edges(edge_index_rev[0], edge_index_rev[1])
    w_mp = jnp.concatenate(
        [edge_weight,
         jnp.zeros((EPAD_ROWS * CW - E,), jnp.float32)]).reshape(
             EPAD_ROWS, CW)

    src12 = jnp.stack([src_mp, src_rv])
    dst12 = jnp.stack([dst_mp, dst_rv])

    agg12 = _sc_conv12(x_measurement, src12, dst12, w_mp)
    movie_x, user_x1 = _tc_combine2(
        agg12, x_measurement, x_demand,
        W_rel1, b_rel1, W_root1, W_rel2, b_rel2, W_root2)
    p3 = _sc_conv3(movie_x, src_rv, dst_rv)
    return _tc_combine3(p3, user_x1, W_rel3, b_rel3, W_root3, W_lin, b_lin)


# speedup vs baseline: 9.7835x; 9.7835x over previous
import functools

import jax
import jax.numpy as jnp
from jax import lax
from jax.experimental import pallas as pl
from jax.experimental.pallas import tpu as pltpu
from jax.experimental.pallas import tpu_sc as plsc

N = 10000
E = 320000
D = 128
O = 64
ACC_ROWS = 10240
EPAD_ROWS = 2560
CW = 128
NB = 2
IG = 16

_MESH = dict(core_axis_name="c", subcore_axis_name="s", num_cores=2,
             num_subcores=16)


def _pad_edges(src, dst):
    pad = EPAD_ROWS * CW - E
    ar = jnp.arange(pad, dtype=jnp.int32)
    src_p = jnp.concatenate([src, ar % N]).reshape(EPAD_ROWS, CW)
    dst_p = jnp.concatenate([dst, N + (ar % (ACC_ROWS - N))]).reshape(
        EPAD_ROWS, CW)
    return src_p, dst_p


def _zero_buf(rows):
    z = jnp.zeros((16,), jnp.float32)

    def body(r, carry):
        for q in range(8):
            rows[0, r, pl.ds(q * 16, 16)] = z
        return carry

    lax.fori_loop(0, 128, body, 0)


def _zero_acc_stripe(rows, acc, s):
    for t in range(5):
        pltpu.sync_copy(rows.at[0], acc.at[pl.ds(s * 640 + t * 128, 128)])


def _scale_rows(rows, b, wbuf, j):
    def grp(g, carry):
        w16 = wbuf[j, pl.ds(g * 16, 16)]
        for i in range(16):
            r = g * 16 + i
            wb = jnp.broadcast_to(w16[i], (16,))
            for q in range(8):
                sl = pl.ds(q * 16, 16)
                rows[b, r, sl] = rows[b, r, sl] * wb
        return carry

    lax.fori_loop(0, 8, grp, 0)


def _edge_loop(x_hbm, stage_idx_fn, src_idx, dst_idx, rows, acc, sems,
               n_chunks, scale_fn):
    def group(g, carry):
        stage_idx_fn(g)
        for b in range(NB):
            pltpu.async_copy(x_hbm.at[src_idx.at[b]], rows.at[b], sems.at[b])

        def inner(io, carry2):
            for b in range(NB):
                j = io * NB + b
                pltpu.make_async_copy(
                    x_hbm.at[src_idx.at[j]], rows.at[b], sems.at[b]).wait()
                scale_fn(rows, b, j)
                pltpu.sync_copy(rows.at[b], acc.at[dst_idx.at[j]], add=True)

                @pl.when(j + NB < IG)
                def _():
                    pltpu.async_copy(
                        x_hbm.at[src_idx.at[j + NB]], rows.at[b], sems.at[b])
            return carry2

        lax.fori_loop(0, IG // NB, inner, 0)
        return carry

    lax.fori_loop(0, n_chunks // IG, group, 0)


@functools.partial(
    pl.kernel,
    out_type=jax.ShapeDtypeStruct((2, N, D), jnp.float32),
    mesh=plsc.VectorSubcoreMesh(**_MESH),
    scratch_types=[
        pltpu.VMEM((IG, CW), jnp.int32),
        pltpu.VMEM((IG, CW), jnp.int32),
        pltpu.VMEM((IG, CW), jnp.float32),
        pltpu.VMEM((NB, CW, D), jnp.float32),
        pltpu.VMEM_SHARED((ACC_ROWS, D), jnp.float32),
        pltpu.SemaphoreType.DMA((NB,)),
    ],
)
def _sc_conv12(x_hbm, src_hbm, dst_hbm, w_hbm, out_hbm,
               src_idx, dst_idx, wbuf, rows, acc, sems):
    c = lax.axis_index("c")
    s = lax.axis_index("s")
    n_chunks = EPAD_ROWS // 16

    _zero_buf(rows)
    _zero_acc_stripe(rows, acc, s)
    plsc.subcore_barrier()

    base = s * n_chunks

    def stage_idx_fn(g):
        rb = base + g * IG
        pltpu.sync_copy(src_hbm.at[c, pl.ds(rb, IG)], src_idx)
        pltpu.sync_copy(dst_hbm.at[c, pl.ds(rb, IG)], dst_idx)

        @pl.when(c == 0)
        def _():
            pltpu.sync_copy(w_hbm.at[pl.ds(rb, IG)], wbuf)

    def scale_fn(rows_, b, j):
        @pl.when(c == 0)
        def _():
            _scale_rows(rows_, b, wbuf, j)

    _edge_loop(x_hbm, stage_idx_fn, src_idx, dst_idx, rows, acc, sems,
               n_chunks, scale_fn)

    plsc.subcore_barrier()
    _copy_out(acc, out_hbm, c, s)


def _copy_out(acc, out_hbm, c, s):
    pltpu.sync_copy(acc.at[pl.ds(s * 624, 624)],
                    out_hbm.at[c, pl.ds(s * 624, 624)])

    @pl.when(s == 15)
    def _():
        pltpu.sync_copy(acc.at[pl.ds(9984, 16)],
                        out_hbm.at[c, pl.ds(9984, 16)])


@functools.partial(
    pl.kernel,
    out_type=jax.ShapeDtypeStruct((2, N, D), jnp.float32),
    mesh=plsc.VectorSubcoreMesh(**_MESH),
    scratch_types=[
        pltpu.VMEM((IG, CW), jnp.int32),
        pltpu.VMEM((IG, CW), jnp.int32),
        pltpu.VMEM((NB, CW, D), jnp.float32),
        pltpu.VMEM_SHARED((ACC_ROWS, D), jnp.float32),
        pltpu.SemaphoreType.DMA((NB,)),
    ],
)
def _sc_conv3(x_hbm, src_hbm, dst_hbm, out_hbm,
              src_idx, dst_idx, rows, acc, sems):
    c = lax.axis_index("c")
    s = lax.axis_index("s")
    n_chunks = EPAD_ROWS // 32

    _zero_buf(rows)
    _zero_acc_stripe(rows, acc, s)
    plsc.subcore_barrier()

    base = (c * 16 + s) * n_chunks

    def stage_idx_fn(g):
        rb = base + g * IG
        pltpu.sync_copy(src_hbm.at[pl.ds(rb, IG)], src_idx)
        pltpu.sync_copy(dst_hbm.at[pl.ds(rb, IG)], dst_idx)

    _edge_loop(x_hbm, stage_idx_fn, src_idx, dst_idx, rows, acc, sems,
               n_chunks, lambda rows_, b, j: None)

    plsc.subcore_barrier()
    _copy_out(acc, out_hbm, c, s)


def _tc_combine2(agg12, x_meas, x_dem, W_rel1, b_rel1, W_root1,
                 W_rel2, b_rel2, W_root2):
    BR = 1000
    grid = (N // BR,)

    def body(agg_ref, xm_ref, xd_ref, wr1_ref, b1_ref, wo1_ref,
             wr2_ref, b2_ref, wo2_ref, mov_ref, usr_ref):
        f32 = jnp.float32
        a1 = agg_ref[0]
        a2 = agg_ref[1]
        m = (jnp.dot(a1, wr1_ref[...], preferred_element_type=f32)
             + b1_ref[...]
             + jnp.dot(xm_ref[...], wo1_ref[...], preferred_element_type=f32))
        u = (jnp.dot(a2, wr2_ref[...], preferred_element_type=f32)
             + b2_ref[...]
             + jnp.dot(xd_ref[...], wo2_ref[...], preferred_element_type=f32))
        mov_ref[...] = jnp.maximum(m, 0.0)
        usr_ref[...] = jnp.maximum(u, 0.0)

    full = lambda shape: pl.BlockSpec(shape, lambda i: (0,) * len(shape))
    return pl.pallas_call(
        body,
        grid=grid,
        in_specs=[
            pl.BlockSpec((2, BR, D), lambda i: (0, i, 0)),
            pl.BlockSpec((BR, D), lambda i: (i, 0)),
            pl.BlockSpec((BR, D), lambda i: (i, 0)),
            full((D, D)), full((1, D)), full((D, D)),
            full((D, D)), full((1, D)), full((D, D)),
        ],
        out_specs=[pl.BlockSpec((BR, D), lambda i: (i, 0)),
                   pl.BlockSpec((BR, D), lambda i: (i, 0))],
        out_shape=[jax.ShapeDtypeStruct((N, D), jnp.float32),
                   jax.ShapeDtypeStruct((N, D), jnp.float32)],
    )(agg12, x_meas, x_dem, W_rel1, b_rel1.reshape(1, D), W_root1,
      W_rel2, b_rel2.reshape(1, D), W_root2)


def _tc_combine3(p3, user_x1, W_rel3, b_rel3, W_root3, W_lin, b_lin):
    BR = 1000
    grid = (N // BR,)

    def body(p3_ref, u1_ref, wr3_ref, b3_ref, wo3_ref, wl_ref, bl_ref,
             out_ref):
        f32 = jnp.float32
        agg3 = p3_ref[0] + p3_ref[1]
        u = (jnp.dot(agg3, wr3_ref[...], preferred_element_type=f32)
             + b3_ref[...]
             + jnp.dot(u1_ref[...], wo3_ref[...], preferred_element_type=f32))
        u = jnp.maximum(u, 0.0)
        out_ref[...] = (jnp.dot(u, wl_ref[...], preferred_element_type=f32)
                        + bl_ref[...])

    full = lambda shape: pl.BlockSpec(shape, lambda i: (0,) * len(shape))
    return pl.pallas_call(
        body,
        grid=grid,
        in_specs=[
            pl.BlockSpec((2, BR, D), lambda i: (0, i, 0)),
            pl.BlockSpec((BR, D), lambda i: (i, 0)),
            full((D, D)), full((1, D)), full((D, D)),
            full((D, O)), full((1, O)),
        ],
        out_specs=pl.BlockSpec((BR, O), lambda i: (i, 0)),
        out_shape=jax.ShapeDtypeStruct((N, O), jnp.float32),
    )(p3, user_x1, W_rel3, b_rel3.reshape(1, D), W_root3,
      W_lin, b_lin.reshape(1, O))


def kernel(x_measurement, x_demand, edge_index_mp, edge_index_rev,
           edge_weight, W_rel1, b_rel1, W_root1, W_rel2, b_rel2, W_root2,
           W_rel3, b_rel3, W_root3, W_lin, b_lin):
    src_mp, dst_mp = _pad_edges(edge_index_mp[0], edge_index_mp[1])
    src_rv, dst_rv = _pad_edges(edge_index_rev[0], edge_index_rev[1])
    w_mp = jnp.concatenate(
        [edge_weight,
         jnp.zeros((EPAD_ROWS * CW - E,), jnp.float32)]).reshape(
             EPAD_ROWS, CW)

    src12 = jnp.stack([src_mp, src_rv])
    dst12 = jnp.stack([dst_mp, dst_rv])

    agg12 = _sc_conv12(x_measurement, src12, dst12, w_mp)
    movie_x, user_x1 = _tc_combine2(
        agg12, x_measurement, x_demand,
        W_rel1, b_rel1, W_root1, W_rel2, b_rel2, W_root2)
    p3 = _sc_conv3(movie_x, src_rv, dst_rv)
    return _tc_combine3(p3, user_x1, W_rel3, b_rel3, W_root3, W_lin, b_lin)
